# baseline (speedup 1.0000x reference)
import functools
import jax
import jax.numpy as jnp
from jax import lax
from jax.experimental import pallas as pl
from jax.experimental.pallas import tpu as pltpu
from jax.experimental.pallas import tpu_sc as plsc

_N = 256
_K = 16
_C = 256
_NC = 2
_NS = 16
_L = 16


def _sc_knn(c_hbm, nbr_hbm, cv, rows_v, sem):
    f32 = jnp.float32
    wid = lax.axis_index("s") * _NC + lax.axis_index("c")
    scene = wid // 4
    base = (wid % 4) * 64
    pltpu.sync_copy(c_hbm.at[scene], cv)
    iota = lax.broadcasted_iota(jnp.int32, (_L,), 0)
    _dn = lax.GatherDimensionNumbers(offset_dims=(), collapsed_slice_dims=(0,),
                                     start_index_map=(0,))

    def bcast(vec, lane):
        return lax.gather(vec, lane[:, None], _dn, (1,),
                          mode=lax.GatherScatterMode.PROMISE_IN_BOUNDS)

    def row_body(r, carry):
        i = base + r
        lane = jnp.full((_L,), i % _L, jnp.int32)
        chunk = (i // _L) * _L
        cxi = bcast(cv[0, pl.ds(chunk, _L)], lane)
        cyi = bcast(cv[1, pl.ds(chunk, _L)], lane)
        czi = bcast(cv[2, pl.ds(chunk, _L)], lane)
        bk = jnp.full((_L,), f32(3e38))
        bv = jnp.zeros((_L,), jnp.int32)
        for j in range(_N // _L):
            s0 = j * _L
            dx = cv[0, s0:s0 + _L] - cxi
            dy = cv[1, s0:s0 + _L] - cyi
            dz = cv[2, s0:s0 + _L] - czi
            dist = (dx * dx + dy * dy) + dz * dz
            idxv = iota + jnp.int32(s0)
            dist = jnp.where(idxv == i, dist + f32(1e10), dist)
            ck, cvv = lax.sort((dist, idxv), num_keys=1)
            rbk = lax.rev(bk, (0,))
            rbv = lax.rev(bv, (0,))
            lt = ck < rbk
            eq = ck == rbk
            tk = jnp.where(lt, ck, rbk)
            tv = jnp.where(lt, cvv, jnp.where(eq, jnp.minimum(cvv, rbv), rbv))
            bk, bv = lax.sort((tk, tv), num_keys=1)
        rows_v[pl.ds(r * _K, _K)] = bv
        return carry

    lax.fori_loop(0, 64, row_body, 0)
    pltpu.sync_copy(rows_v, nbr_hbm.at[pl.ds(wid * 64 * _K, 64 * _K)])


def _knn_indices(center):
    B = center.shape[0]
    c3 = jnp.transpose(center, (0, 2, 1))
    mesh = plsc.VectorSubcoreMesh(core_axis_name="c", subcore_axis_name="s")
    k = functools.partial(
        pl.kernel, mesh=mesh,
        out_type=jax.ShapeDtypeStruct((B * _N * _K,), jnp.int32),
        scratch_types=[pltpu.VMEM((3, _N), jnp.float32),
                       pltpu.VMEM((64 * _K,), jnp.int32),
                       pltpu.SemaphoreType.DMA],
        compiler_params=pltpu.CompilerParams(needs_layout_passes=False),
    )(_sc_knn)
    return k(c3).reshape(B, _N, _K)


def _tc_kernel(x_ref, mask_ref, nbr_ref,
               Wd1_ref, Wb1_ref, b11_ref, W12_ref, b12_ref,
               Wd2_ref, Wb2_ref, b21_ref, W22_ref, b22_ref,
               out_ref):
    f32 = jnp.float32
    bf16 = jnp.bfloat16
    col_iota = jax.lax.broadcasted_iota(jnp.int32, (_N, _N), 1)
    nbr = nbr_ref[0]

    def edgeconv(xin, Wd_ref, Wb_ref, b1_ref, W2_ref, b2_ref):
        P = jnp.dot(xin, Wd_ref[...], preferred_element_type=f32) + b1_ref[...]
        Q = jnp.dot(xin, Wb_ref[...], preferred_element_type=f32).astype(bf16)
        W2 = W2_ref[...]
        acc = jnp.full((_N, _C), -jnp.inf, f32)
        for t in range(_K):
            onehot = jnp.where(col_iota == nbr[:, t:t + 1],
                               f32(1.0), f32(0.0)).astype(bf16)
            G = jnp.dot(onehot, Q, preferred_element_type=f32)
            H = jnp.maximum(P + G, f32(0.0))
            O = jnp.dot(H, W2, preferred_element_type=f32)
            acc = jnp.maximum(acc, O)
        return acc + b2_ref[...]

    x = x_ref[0]
    h = edgeconv(x, Wd1_ref, Wb1_ref, b11_ref, W12_ref, b12_ref)
    h = jnp.maximum(h, f32(0.0))
    h = edgeconv(h, Wd2_ref, Wb2_ref, b21_ref, W22_ref, b22_ref)
    mask = mask_ref[0]
    out_ref[0] = jnp.where(mask > f32(0.0), h, x)


def kernel(object_feat, bbox_mask, center, W11, b11, W12, b12, W21, b21, W22, b22):
    B = object_feat.shape[0]
    nbr = _knn_indices(center)
    mask3 = bbox_mask.reshape(B, _N, 1)
    Wd1 = W11[:_C] - W11[_C:]
    Wd2 = W21[:_C] - W21[_C:]

    def w_spec(shape):
        return pl.BlockSpec(shape, lambda g: (0,) * len(shape))

    def b_spec(shape):
        return pl.BlockSpec(shape, lambda g: (g, 0, 0))

    out = pl.pallas_call(
        _tc_kernel,
        grid=(B,),
        in_specs=[
            b_spec((1, _N, _C)),
            b_spec((1, _N, 1)),
            b_spec((1, _N, _K)),
            w_spec((_C, _C)), w_spec((_C, _C)), w_spec((1, _C)),
            w_spec((_C, _C)), w_spec((1, _C)),
            w_spec((_C, _C)), w_spec((_C, _C)), w_spec((1, _C)),
            w_spec((_C, _C)), w_spec((1, _C)),
        ],
        out_specs=b_spec((1, _N, _C)),
        out_shape=jax.ShapeDtypeStruct((B, _N, _C), jnp.float32),
    )(object_feat, mask3, nbr,
      Wd1, W11[_C:], b11.reshape(1, _C), W12, b12.reshape(1, _C),
      Wd2, W21[_C:], b21.reshape(1, _C), W22, b22.reshape(1, _C))
    return out


# speedup vs baseline: 1.6632x; 1.6632x over previous
import jax
import jax.numpy as jnp
from jax.experimental import pallas as pl
from jax.experimental.pallas import tpu as pltpu

_N = 256
_K = 16
_C = 256


def _scene_kernel(x_ref, mask_ref, ccol_ref, crow_ref,
                  Wd1_ref, Wb1_ref, b11_ref, W12_ref, b12_ref,
                  Wd2_ref, Wb2_ref, b21_ref, W22_ref, b22_ref,
                  out_ref, A_ref, d_ref):
    f32 = jnp.float32
    col_iota = jax.lax.broadcasted_iota(jnp.int32, (_N, _N), 1)
    row_iota = jax.lax.broadcasted_iota(jnp.int32, (_N, _N), 0)

    bf16 = jnp.bfloat16

    def edgeconv(xin_b, Wd_ref, Wb_ref, b1_ref, W2_ref, b2_ref):
        P = jnp.dot(xin_b, Wd_ref[...], preferred_element_type=f32) + b1_ref[...]
        Q = jnp.dot(xin_b, Wb_ref[...], preferred_element_type=f32).astype(bf16)
        W2 = W2_ref[...]
        acc = jnp.full((_N, _C), -jnp.inf, f32)
        for t in range(_K):
            G = jnp.dot(A_ref[t * _N:(t + 1) * _N, :], Q,
                        preferred_element_type=f32)
            H = jnp.maximum(P + G, f32(0.0))
            O = jnp.dot(H, W2, preferred_element_type=f32)
            acc = jnp.maximum(acc, O)
        return acc + b2_ref[...]

    x = x_ref[0]
    h = edgeconv(x, Wd1_ref, Wb1_ref, b11_ref, W12_ref, b12_ref)
    h = jnp.maximum(h, f32(0.0))
    h = edgeconv(h, Wd2_ref, Wb2_ref, b21_ref, W22_ref, b22_ref)
    mask = mask_ref[0]
    out_ref[0] = jnp.where(mask > f32(0.0), h, x)

    ccol = ccol_ref[0]
    crow = crow_ref[0]
    dx = ccol[:, 0:1] - crow[0:1, :]
    dy = ccol[:, 1:2] - crow[1:2, :]
    dz = ccol[:, 2:3] - crow[2:3, :]
    d = (dx * dx + dy * dy) + dz * dz
    d = d + jnp.where(row_iota == col_iota, f32(1e10), f32(0.0))
    d_ref[...] = d
    col_f = col_iota.astype(f32)
    for t in range(_K):
        dcur = d_ref[...]
        m = jnp.min(dcur, axis=1, keepdims=True)
        tie = jnp.where(dcur == m, col_f, f32(_N))
        idx = jnp.min(tie, axis=1, keepdims=True)
        sel = col_f == idx
        A_ref[t * _N:(t + 1) * _N, :] = jnp.where(sel, f32(1.0), f32(0.0)).astype(jnp.bfloat16)
        d_ref[...] = jnp.where(sel, f32(3e38), dcur)


def kernel(object_feat, bbox_mask, center, W11, b11, W12, b12, W21, b21, W22, b22):
    B = object_feat.shape[0]
    cpad = jnp.pad(center, ((0, 0), (0, 0), (0, 5)))
    crow = jnp.transpose(cpad, (0, 2, 1))
    mask3 = bbox_mask.reshape(B, _N, 1)
    Wd1 = W11[:_C] - W11[_C:]
    Wd2 = W21[:_C] - W21[_C:]

    def w_spec(shape):
        return pl.BlockSpec(shape, lambda g: (0,) * len(shape))

    def prev_spec(shape):
        return pl.BlockSpec(shape, lambda g: (jnp.maximum(g - 1, 0), 0, 0))

    def cur_spec(shape):
        return pl.BlockSpec(shape, lambda g: (jnp.minimum(g, B - 1), 0, 0))

    out = pl.pallas_call(
        _scene_kernel,
        grid=(B + 1,),
        in_specs=[
            prev_spec((1, _N, _C)),
            prev_spec((1, _N, 1)),
            cur_spec((1, _N, 8)),
            cur_spec((1, 8, _N)),
            w_spec((_C, _C)), w_spec((_C, _C)), w_spec((1, _C)),
            w_spec((_C, _C)), w_spec((1, _C)),
            w_spec((_C, _C)), w_spec((_C, _C)), w_spec((1, _C)),
            w_spec((_C, _C)), w_spec((1, _C)),
        ],
        out_specs=prev_spec((1, _N, _C)),
        out_shape=jax.ShapeDtypeStruct((B, _N, _C), jnp.float32),
        scratch_shapes=[pltpu.VMEM((_K * _N, _N), jnp.bfloat16),
                        pltpu.VMEM((_N, _N), jnp.float32)],
    )(object_feat, mask3, cpad, crow,
      Wd1, W11[_C:], b11.reshape(1, _C), W12, b12.reshape(1, _C),
      Wd2, W21[_C:], b21.reshape(1, _C), W22, b22.reshape(1, _C))
    return out
